# initial kernel scaffold (unmeasured)
import jax
import jax.numpy as jnp
from jax import lax
from jax.experimental import pallas as pl
from jax.experimental.pallas import tpu as pltpu

N_DEV = 32


def kernel(x, w_mat):
    M, _ = x.shape
    _, N = w_mat.shape
    m_per = M // N_DEV

    def body(x_ref, w_ref, out_ref, comm_ref, send_sems, recv_sems,
             credit_sem, amax_ref, amax_send_sems, amax_recv_sems):
        p = lax.axis_index("i")
        left = lax.rem(p + N_DEV - 1, N_DEV)
        right = lax.rem(p + 1, N_DEV)

        barrier = pltpu.get_barrier_semaphore()
        for nbr in (left, right):
            pl.semaphore_signal(barrier, inc=1, device_id=(nbr,),
                                device_id_type=pl.DeviceIdType.MESH)
        pl.semaphore_wait(barrier, 2)

        def partial_chunk(c):
            xa = x_ref[pl.ds(c * m_per, m_per), :]
            return lax.dot_general(
                xa, w_ref[:, :], (((1,), (0,)), ((), ())),
                preferred_element_type=jnp.float32,
                precision=lax.Precision.HIGHEST)

        comm_ref[0] = partial_chunk(left)

        for s in range(N_DEV - 1):
            send_slot = s % 2
            recv_slot = (s + 1) % 2
            if s >= 1:
                pl.semaphore_wait(credit_sem, 1)
            rdma = pltpu.make_async_remote_copy(
                src_ref=comm_ref.at[send_slot],
                dst_ref=comm_ref.at[recv_slot],
                send_sem=send_sems.at[send_slot],
                recv_sem=recv_sems.at[recv_slot],
                device_id=(right,),
                device_id_type=pl.DeviceIdType.MESH,
            )
            rdma.start()
            pc = partial_chunk(lax.rem(p + 2 * N_DEV - s - 2, N_DEV))
            rdma.wait()
            if s < N_DEV - 2:
                pl.semaphore_signal(credit_sem, inc=1, device_id=(left,),
                                    device_id_type=pl.DeviceIdType.MESH)
            comm_ref[recv_slot] = comm_ref[recv_slot] + pc

        y = jnp.maximum(comm_ref[(N_DEV - 1) % 2], 0.0)
        out_ref[:, :] = y

        amax_ref[pl.ds(p, 1), :] = jnp.broadcast_to(jnp.max(y), (1, 128))
        sends = []
        for j in range(1, N_DEV):
            tgt = lax.rem(p + j, N_DEV)
            r = pltpu.make_async_remote_copy(
                src_ref=amax_ref.at[pl.ds(p, 1)],
                dst_ref=amax_ref.at[pl.ds(p, 1)],
                send_sem=amax_send_sems.at[tgt],
                recv_sem=amax_recv_sems.at[p],
                device_id=(tgt,),
                device_id_type=pl.DeviceIdType.MESH,
            )
            r.start()
            sends.append(r)
        for r in sends:
            r.wait_send()
        for j in range(1, N_DEV):
            src = lax.rem(p + N_DEV - j, N_DEV)
            rr = pltpu.make_async_remote_copy(
                src_ref=amax_ref.at[pl.ds(p, 1)],
                dst_ref=amax_ref.at[pl.ds(src, 1)],
                send_sem=amax_send_sems.at[src],
                recv_sem=amax_recv_sems.at[src],
                device_id=(src,),
                device_id_type=pl.DeviceIdType.MESH,
            )
            rr.wait_recv()

        gmax = jnp.max(amax_ref[:, :])
        scale = gmax / 448.0
        ys = jnp.minimum(out_ref[:, :] / scale, 448.0)
        q = ys.astype(jnp.float8_e4m3fn).astype(jnp.float32)
        out_ref[:, :] = q * scale

    return pl.pallas_call(
        body,
        out_shape=jax.ShapeDtypeStruct((m_per, N), jnp.float32),
        in_specs=[pl.BlockSpec(memory_space=pltpu.VMEM),
                  pl.BlockSpec(memory_space=pltpu.VMEM)],
        out_specs=pl.BlockSpec(memory_space=pltpu.VMEM),
        scratch_shapes=[
            pltpu.VMEM((2, m_per, N), jnp.float32),
            pltpu.SemaphoreType.DMA((2,)),
            pltpu.SemaphoreType.DMA((2,)),
            pltpu.SemaphoreType.REGULAR,
            pltpu.VMEM((N_DEV, 128), jnp.float32),
            pltpu.SemaphoreType.DMA((N_DEV,)),
            pltpu.SemaphoreType.DMA((N_DEV,)),
        ],
        compiler_params=pltpu.CompilerParams(collective_id=0),
    )(x, w_mat)


# baseline (device time: 1629647 ns/iter reference)
import jax
import jax.numpy as jnp
from jax import lax
from jax.experimental import pallas as pl
from jax.experimental.pallas import tpu as pltpu

N_DEV = 32


def kernel(x, w_mat):
    M, _ = x.shape
    _, N = w_mat.shape
    m_per = M // N_DEV

    def body(x_ref, w_ref, out_ref, comm_ref, send_sems, recv_sems,
             credit_sem, amax_ref, amax_send_sems, amax_recv_sems):
        p = lax.axis_index("i")
        left = lax.rem(p + N_DEV - 1, N_DEV)
        right = lax.rem(p + 1, N_DEV)

        barrier = pltpu.get_barrier_semaphore()
        for nbr in (left, right):
            pl.semaphore_signal(barrier, inc=1, device_id=(nbr,),
                                device_id_type=pl.DeviceIdType.MESH)
        pl.semaphore_wait(barrier, 2)

        def partial_chunk(c):
            xa = x_ref[pl.ds(c * m_per, m_per), :]
            return lax.dot_general(
                xa, w_ref[:, :], (((1,), (0,)), ((), ())),
                preferred_element_type=jnp.float32,
                precision=lax.Precision.HIGHEST)

        comm_ref[0] = partial_chunk(left)

        def step(s, carry):
            send_slot = lax.rem(s, 2)
            recv_slot = lax.rem(s + 1, 2)

            @pl.when(s >= 1)
            def _():
                pl.semaphore_wait(credit_sem, 1)

            rdma = pltpu.make_async_remote_copy(
                src_ref=comm_ref.at[send_slot],
                dst_ref=comm_ref.at[recv_slot],
                send_sem=send_sems.at[send_slot],
                recv_sem=recv_sems.at[recv_slot],
                device_id=(right,),
                device_id_type=pl.DeviceIdType.MESH,
            )
            rdma.start()
            pc = partial_chunk(lax.rem(p + 2 * N_DEV - s - 2, N_DEV))
            rdma.wait()

            @pl.when(s < N_DEV - 2)
            def _():
                pl.semaphore_signal(credit_sem, inc=1, device_id=(left,),
                                    device_id_type=pl.DeviceIdType.MESH)

            comm_ref[recv_slot] = comm_ref[recv_slot] + pc
            return carry

        lax.fori_loop(0, N_DEV - 1, step, 0)

        y = jnp.maximum(comm_ref[(N_DEV - 1) % 2], 0.0)
        out_ref[:, :] = y

        amax_ref[pl.ds(p, 1), :] = jnp.broadcast_to(jnp.max(y), (1, 128))
        sends = []
        for j in range(1, N_DEV):
            tgt = lax.rem(p + j, N_DEV)
            r = pltpu.make_async_remote_copy(
                src_ref=amax_ref.at[pl.ds(p, 1)],
                dst_ref=amax_ref.at[pl.ds(p, 1)],
                send_sem=amax_send_sems.at[tgt],
                recv_sem=amax_recv_sems.at[p],
                device_id=(tgt,),
                device_id_type=pl.DeviceIdType.MESH,
            )
            r.start()
            sends.append(r)
        for r in sends:
            r.wait_send()
        for j in range(1, N_DEV):
            src = lax.rem(p + N_DEV - j, N_DEV)
            rr = pltpu.make_async_remote_copy(
                src_ref=amax_ref.at[pl.ds(p, 1)],
                dst_ref=amax_ref.at[pl.ds(src, 1)],
                send_sem=amax_send_sems.at[src],
                recv_sem=amax_recv_sems.at[src],
                device_id=(src,),
                device_id_type=pl.DeviceIdType.MESH,
            )
            rr.wait_recv()

        gmax = jnp.max(amax_ref[:, :])
        scale = gmax / 448.0
        ys = jnp.minimum(out_ref[:, :] / scale, 448.0)
        q = ys.astype(jnp.float8_e4m3fn).astype(jnp.float32)
        out_ref[:, :] = q * scale

    return pl.pallas_call(
        body,
        out_shape=jax.ShapeDtypeStruct((m_per, N), jnp.float32),
        in_specs=[pl.BlockSpec(memory_space=pltpu.VMEM),
                  pl.BlockSpec(memory_space=pltpu.VMEM)],
        out_specs=pl.BlockSpec(memory_space=pltpu.VMEM),
        scratch_shapes=[
            pltpu.VMEM((2, m_per, N), jnp.float32),
            pltpu.SemaphoreType.DMA((2,)),
            pltpu.SemaphoreType.DMA((2,)),
            pltpu.SemaphoreType.REGULAR,
            pltpu.VMEM((N_DEV, 128), jnp.float32),
            pltpu.SemaphoreType.DMA((N_DEV,)),
            pltpu.SemaphoreType.DMA((N_DEV,)),
        ],
        compiler_params=pltpu.CompilerParams(collective_id=0),
    )(x, w_mat)


# device time: 810166 ns/iter; 2.0115x vs baseline; 2.0115x over previous
import jax
import jax.numpy as jnp
from jax import lax
from jax.experimental import pallas as pl
from jax.experimental.pallas import tpu as pltpu

N_DEV = 32


def _ring_tables():
    coords_list = []
    for z in range(4):
        for yi in range(4):
            xs = (0, 1) if yi % 2 == 0 else (1, 0)
            for xx in xs:
                coords_list.append((xx, yi, z))
    log_of = {c: i for i, c in enumerate(coords_list)}

    path = []
    for y in range(4):
        zs = range(4) if y % 2 == 0 else range(3, -1, -1)
        path.extend((y, z) for z in zs)
    cyc = [(0, y, z) for (y, z) in path] + [(1, y, z) for (y, z) in reversed(path)]

    ring = [log_of[c] for c in cyc]
    pos = [0] * N_DEV
    nxt = [0] * N_DEV
    prv = [0] * N_DEV
    for r, l in enumerate(ring):
        pos[l] = r
        nxt[l] = ring[(r + 1) % N_DEV]
        prv[l] = ring[(r - 1) % N_DEV]
    return ring, pos, nxt, prv


_RING, _POS, _NXT, _PRV = _ring_tables()


def kernel(x, w_mat):
    M, _ = x.shape
    _, N = w_mat.shape
    m_per = M // N_DEV
    half = N // 2

    def body(x_ref, w_ref, ring_ref, pos_ref, nxt_ref, prv_ref,
             out_ref, comm_r, comm_l, send_r, recv_r,
             send_l, recv_l, credit_r, credit_l,
             amax_ref, amax_ss, amax_rs):
        p = lax.axis_index("i")

        def ring_at(r):
            return ring_ref[lax.rem(r + 2 * N_DEV, N_DEV)]

        rpos = pos_ref[p]
        right = nxt_ref[p]
        left = prv_ref[p]

        barrier = pltpu.get_barrier_semaphore()
        for nbr in (left, right):
            pl.semaphore_signal(barrier, inc=1, device_id=(nbr,),
                                device_id_type=pl.DeviceIdType.MESH)
        pl.semaphore_wait(barrier, 2)

        def pchunk(c, h):
            xa = x_ref[pl.ds(c * m_per, m_per), :]
            wv = w_ref[:, h * half:(h + 1) * half]
            return lax.dot_general(
                xa, wv, (((1,), (0,)), ((), ())),
                preferred_element_type=jnp.float32,
                precision=lax.Precision.HIGHEST)

        comm_r[0] = pchunk(left, 0)
        comm_l[0] = pchunk(right, 1)

        def step(s, carry):
            snd = lax.rem(s, 2)
            rcv = lax.rem(s + 1, 2)

            @pl.when(s >= 1)
            def _():
                pl.semaphore_wait(credit_r, 1)
                pl.semaphore_wait(credit_l, 1)

            rdma_r = pltpu.make_async_remote_copy(
                src_ref=comm_r.at[snd], dst_ref=comm_r.at[rcv],
                send_sem=send_r.at[snd], recv_sem=recv_r.at[rcv],
                device_id=(right,), device_id_type=pl.DeviceIdType.MESH)
            rdma_l = pltpu.make_async_remote_copy(
                src_ref=comm_l.at[snd], dst_ref=comm_l.at[rcv],
                send_sem=send_l.at[snd], recv_sem=recv_l.at[rcv],
                device_id=(left,), device_id_type=pl.DeviceIdType.MESH)
            rdma_r.start()
            rdma_l.start()

            pc_r = pchunk(ring_at(rpos - s - 2), 0)
            pc_l = pchunk(ring_at(rpos + s + 2), 1)

            rdma_r.wait()
            comm_r[rcv] = comm_r[rcv] + pc_r
            rdma_l.wait()
            comm_l[rcv] = comm_l[rcv] + pc_l

            @pl.when(s < N_DEV - 2)
            def _():
                pl.semaphore_signal(credit_r, inc=1, device_id=(left,),
                                    device_id_type=pl.DeviceIdType.MESH)
                pl.semaphore_signal(credit_l, inc=1, device_id=(right,),
                                    device_id_type=pl.DeviceIdType.MESH)

            return carry

        lax.fori_loop(0, N_DEV - 1, step, 0)

        y_r = jnp.maximum(comm_r[(N_DEV - 1) % 2], 0.0)
        y_l = jnp.maximum(comm_l[(N_DEV - 1) % 2], 0.0)
        out_ref[:, :half] = y_r
        out_ref[:, half:] = y_l

        m = jnp.maximum(jnp.max(y_r), jnp.max(y_l))
        amax_ref[pl.ds(p, 1), :] = jnp.broadcast_to(m, (1, 128))
        sends = []
        for j in range(1, N_DEV):
            tgt = lax.rem(p + j, N_DEV)
            r = pltpu.make_async_remote_copy(
                src_ref=amax_ref.at[pl.ds(p, 1)],
                dst_ref=amax_ref.at[pl.ds(p, 1)],
                send_sem=amax_ss.at[tgt],
                recv_sem=amax_rs.at[p],
                device_id=(tgt,),
                device_id_type=pl.DeviceIdType.MESH)
            r.start()
            sends.append(r)
        for r in sends:
            r.wait_send()
        for j in range(1, N_DEV):
            src = lax.rem(p + N_DEV - j, N_DEV)
            rr = pltpu.make_async_remote_copy(
                src_ref=amax_ref.at[pl.ds(p, 1)],
                dst_ref=amax_ref.at[pl.ds(src, 1)],
                send_sem=amax_ss.at[src],
                recv_sem=amax_rs.at[src],
                device_id=(src,),
                device_id_type=pl.DeviceIdType.MESH)
            rr.wait_recv()

        gmax = jnp.max(amax_ref[:, :])
        scale = gmax / 448.0
        ys = jnp.minimum(out_ref[:, :] / scale, 448.0)
        q = ys.astype(jnp.float8_e4m3fn).astype(jnp.float32)
        out_ref[:, :] = q * scale

    return pl.pallas_call(
        body,
        out_shape=jax.ShapeDtypeStruct((m_per, N), jnp.float32),
        in_specs=[pl.BlockSpec(memory_space=pltpu.VMEM),
                  pl.BlockSpec(memory_space=pltpu.VMEM),
                  pl.BlockSpec(memory_space=pltpu.SMEM),
                  pl.BlockSpec(memory_space=pltpu.SMEM),
                  pl.BlockSpec(memory_space=pltpu.SMEM),
                  pl.BlockSpec(memory_space=pltpu.SMEM)],
        out_specs=pl.BlockSpec(memory_space=pltpu.VMEM),
        scratch_shapes=[
            pltpu.VMEM((2, m_per, half), jnp.float32),
            pltpu.VMEM((2, m_per, half), jnp.float32),
            pltpu.SemaphoreType.DMA((2,)),
            pltpu.SemaphoreType.DMA((2,)),
            pltpu.SemaphoreType.DMA((2,)),
            pltpu.SemaphoreType.DMA((2,)),
            pltpu.SemaphoreType.REGULAR,
            pltpu.SemaphoreType.REGULAR,
            pltpu.VMEM((N_DEV, 128), jnp.float32),
            pltpu.SemaphoreType.DMA((N_DEV,)),
            pltpu.SemaphoreType.DMA((N_DEV,)),
        ],
        compiler_params=pltpu.CompilerParams(collective_id=0),
    )(x, w_mat,
      jnp.asarray(_RING, dtype=jnp.int32),
      jnp.asarray(_POS, dtype=jnp.int32),
      jnp.asarray(_NXT, dtype=jnp.int32),
      jnp.asarray(_PRV, dtype=jnp.int32))


# device time: 725801 ns/iter; 2.2453x vs baseline; 1.1162x over previous
import jax
import jax.numpy as jnp
from jax import lax
from jax.experimental import pallas as pl
from jax.experimental.pallas import tpu as pltpu

N_DEV = 32


def _ring_tables():
    coords_list = []
    for z in range(4):
        for yi in range(4):
            xs = (0, 1) if yi % 2 == 0 else (1, 0)
            for xx in xs:
                coords_list.append((xx, yi, z))
    log_of = {c: i for i, c in enumerate(coords_list)}

    path = []
    for y in range(4):
        zs = range(4) if y % 2 == 0 else range(3, -1, -1)
        path.extend((y, z) for z in zs)
    cyc = [(0, y, z) for (y, z) in path] + [(1, y, z) for (y, z) in reversed(path)]

    ring = [log_of[c] for c in cyc]
    pos = [0] * N_DEV
    nxt = [0] * N_DEV
    prv = [0] * N_DEV
    for r, l in enumerate(ring):
        pos[l] = r
        nxt[l] = ring[(r + 1) % N_DEV]
        prv[l] = ring[(r - 1) % N_DEV]
    return ring, pos, nxt, prv


_RING, _POS, _NXT, _PRV = _ring_tables()


def kernel(x, w_mat):
    M, _ = x.shape
    _, N = w_mat.shape
    m_per = M // N_DEV
    half = N // 2
    qtr = N // 4

    def body(x_ref, w_ref, ring_ref, pos_ref, nxt_ref, prv_ref,
             out_ref,
             comm_ra, comm_rb, comm_la, comm_lb,
             ss_ra, rs_ra, ss_rb, rs_rb,
             ss_la, rs_la, ss_lb, rs_lb,
             cr_ra, cr_rb, cr_la, cr_lb,
             amax_ref, amax_ss, amax_rs):
        p = lax.axis_index("i")

        def ring_at(r):
            return ring_ref[lax.rem(r + 2 * N_DEV, N_DEV)]

        rpos = pos_ref[p]
        right = nxt_ref[p]
        left = prv_ref[p]

        barrier = pltpu.get_barrier_semaphore()
        for nbr in (left, right):
            pl.semaphore_signal(barrier, inc=1, device_id=(nbr,),
                                device_id_type=pl.DeviceIdType.MESH)
        pl.semaphore_wait(barrier, 2)

        def pchunk(c, h):
            xa = x_ref[pl.ds(c * m_per, m_per), :]
            wv = w_ref[:, h * half:(h + 1) * half]
            return lax.dot_general(
                xa, wv, (((1,), (0,)), ((), ())),
                preferred_element_type=jnp.float32,
                precision=lax.Precision.HIGHEST)

        pipes = (
            (comm_ra, ss_ra, rs_ra, cr_ra, right, left),
            (comm_la, ss_la, rs_la, cr_la, left, right),
            (comm_rb, ss_rb, rs_rb, cr_rb, right, left),
            (comm_lb, ss_lb, rs_lb, cr_lb, left, right),
        )

        def mk(pipe, snd, rcv, dev):
            comm, ss, rs, _, _, _ = pipe
            return pltpu.make_async_remote_copy(
                src_ref=comm.at[snd], dst_ref=comm.at[rcv],
                send_sem=ss.at[snd], recv_sem=rs.at[rcv],
                device_id=(dev,), device_id_type=pl.DeviceIdType.MESH)

        pc0_r = pchunk(left, 0)
        pc0_l = pchunk(right, 1)
        comm_ra[0] = pc0_r[:, :qtr]
        comm_rb[0] = pc0_r[:, qtr:]
        comm_la[0] = pc0_l[:, :qtr]
        comm_lb[0] = pc0_l[:, qtr:]
        for pipe in pipes:
            mk(pipe, 0, 1, pipe[4]).start()

        def step(s, carry):
            snd = lax.rem(s, 2)
            rcv = lax.rem(s + 1, 2)

            pc_r = pchunk(ring_at(rpos - s - 2), 0)
            pc_l = pchunk(ring_at(rpos + s + 2), 1)
            quarters = (pc_r[:, :qtr], pc_l[:, :qtr],
                        pc_r[:, qtr:], pc_l[:, qtr:])

            for pipe, pc in zip(pipes, quarters):
                comm, ss, rs, credit, down, up = pipe
                mk(pipe, rcv, rcv, up).wait_recv()
                comm[rcv] = comm[rcv] + pc
                mk(pipe, snd, snd, down).wait_send()
                pl.semaphore_signal(credit, inc=1, device_id=(up,),
                                    device_id_type=pl.DeviceIdType.MESH)

                @pl.when(s < N_DEV - 2)
                def _():
                    pl.semaphore_wait(credit, 1)
                    mk(pipe, rcv, snd, down).start()

            return carry

        lax.fori_loop(0, N_DEV - 1, step, 0)

        for pipe in pipes:
            pl.semaphore_wait(pipe[3], 1)

        fin = (N_DEV - 1) % 2
        y_r = jnp.maximum(jnp.concatenate(
            [comm_ra[fin], comm_rb[fin]], axis=1), 0.0)
        y_l = jnp.maximum(jnp.concatenate(
            [comm_la[fin], comm_lb[fin]], axis=1), 0.0)
        out_ref[:, :half] = y_r
        out_ref[:, half:] = y_l

        m = jnp.maximum(jnp.max(y_r), jnp.max(y_l))
        amax_ref[pl.ds(p, 1), :] = jnp.broadcast_to(m, (1, 128))
        sends = []
        for j in range(1, N_DEV):
            tgt = lax.rem(p + j, N_DEV)
            r = pltpu.make_async_remote_copy(
                src_ref=amax_ref.at[pl.ds(p, 1)],
                dst_ref=amax_ref.at[pl.ds(p, 1)],
                send_sem=amax_ss.at[tgt],
                recv_sem=amax_rs.at[p],
                device_id=(tgt,),
                device_id_type=pl.DeviceIdType.MESH)
            r.start()
            sends.append(r)
        for r in sends:
            r.wait_send()
        for j in range(1, N_DEV):
            src = lax.rem(p + N_DEV - j, N_DEV)
            rr = pltpu.make_async_remote_copy(
                src_ref=amax_ref.at[pl.ds(p, 1)],
                dst_ref=amax_ref.at[pl.ds(src, 1)],
                send_sem=amax_ss.at[src],
                recv_sem=amax_rs.at[src],
                device_id=(src,),
                device_id_type=pl.DeviceIdType.MESH)
            rr.wait_recv()

        gmax = jnp.max(amax_ref[:, :])
        scale = gmax / 448.0
        ys = jnp.minimum(out_ref[:, :] / scale, 448.0)
        q = ys.astype(jnp.float8_e4m3fn).astype(jnp.float32)
        out_ref[:, :] = q * scale

    dma2 = pltpu.SemaphoreType.DMA((2,))
    return pl.pallas_call(
        body,
        out_shape=jax.ShapeDtypeStruct((m_per, N), jnp.float32),
        in_specs=[pl.BlockSpec(memory_space=pltpu.VMEM),
                  pl.BlockSpec(memory_space=pltpu.VMEM),
                  pl.BlockSpec(memory_space=pltpu.SMEM),
                  pl.BlockSpec(memory_space=pltpu.SMEM),
                  pl.BlockSpec(memory_space=pltpu.SMEM),
                  pl.BlockSpec(memory_space=pltpu.SMEM)],
        out_specs=pl.BlockSpec(memory_space=pltpu.VMEM),
        scratch_shapes=[
            pltpu.VMEM((2, m_per, qtr), jnp.float32),
            pltpu.VMEM((2, m_per, qtr), jnp.float32),
            pltpu.VMEM((2, m_per, qtr), jnp.float32),
            pltpu.VMEM((2, m_per, qtr), jnp.float32),
            dma2, dma2,
            dma2, dma2,
            dma2, dma2,
            dma2, dma2,
            pltpu.SemaphoreType.REGULAR,
            pltpu.SemaphoreType.REGULAR,
            pltpu.SemaphoreType.REGULAR,
            pltpu.SemaphoreType.REGULAR,
            pltpu.VMEM((N_DEV, 128), jnp.float32),
            pltpu.SemaphoreType.DMA((N_DEV,)),
            pltpu.SemaphoreType.DMA((N_DEV,)),
        ],
        compiler_params=pltpu.CompilerParams(collective_id=0),
    )(x, w_mat,
      jnp.asarray(_RING, dtype=jnp.int32),
      jnp.asarray(_POS, dtype=jnp.int32),
      jnp.asarray(_NXT, dtype=jnp.int32),
      jnp.asarray(_PRV, dtype=jnp.int32))
